# Initial kernel scaffold; baseline (speedup 1.0000x reference)
import jax
import jax.numpy as jnp
from jax.experimental import pallas as pl


def kernel(pred_boxes, pred_scores, pred_cls_inds):
    raise NotImplementedError("write your pallas kernel here")


# speedup vs baseline: 151.5946x; 151.5946x over previous
import jax
import jax.numpy as jnp
from jax import lax
from jax.experimental import pallas as pl
from jax.experimental.pallas import tpu as pltpu

_C = 20
_K = 10
_MAXD = _C * _K
_IMG = 512.0
_SCORE_THR = 0.3
_IOU_THR = 0.5
_NEG = float("-inf")


def _nms_body(b_ref, s_ref, c_ref, o_ref, S_ref):
    NP = s_ref.shape[-1]
    X1 = jnp.clip(b_ref[0, 0:1, :], 0.0, _IMG)
    Y1 = jnp.clip(b_ref[0, 1:2, :], 0.0, _IMG)
    X2 = jnp.clip(b_ref[0, 2:3, :], 0.0, _IMG)
    Y2 = jnp.clip(b_ref[0, 3:4, :], 0.0, _IMG)
    sc = s_ref[0]
    cl = c_ref[0]
    AREA = jnp.maximum(X2 - X1, 0.0) * jnp.maximum(Y2 - Y1, 0.0)
    ok = ((X2 - X1) >= 0.01) & ((Y2 - Y1) >= 0.01)
    base = (sc > _SCORE_THR) & ok

    ciota = lax.broadcasted_iota(jnp.int32, (_C, NP), 0)
    iN = lax.broadcasted_iota(jnp.int32, (_C, NP), 1)
    kiota = lax.broadcasted_iota(jnp.int32, (_C, _K), 1)
    S_ref[...] = jnp.where(base & (cl == ciota), jnp.broadcast_to(sc, (_C, NP)),
                           _NEG)

    zK = jnp.zeros((_C, _K), jnp.float32)

    def round_body(r, carry):
        KX1, KY1, KX2, KY2, KS, KM = carry
        S = S_ref[...]
        m = jnp.max(S, axis=1, keepdims=True)
        alive = m > _NEG
        idx = jnp.min(jnp.where(S == m, iN, NP), axis=1, keepdims=True)
        oh = iN == idx
        bx1 = jnp.sum(jnp.where(oh, X1, 0.0), axis=1, keepdims=True)
        by1 = jnp.sum(jnp.where(oh, Y1, 0.0), axis=1, keepdims=True)
        bx2 = jnp.sum(jnp.where(oh, X2, 0.0), axis=1, keepdims=True)
        by2 = jnp.sum(jnp.where(oh, Y2, 0.0), axis=1, keepdims=True)
        barea = jnp.maximum(bx2 - bx1, 0.0) * jnp.maximum(by2 - by1, 0.0)
        xx1 = jnp.maximum(bx1, X1)
        yy1 = jnp.maximum(by1, Y1)
        xx2 = jnp.minimum(bx2, X2)
        yy2 = jnp.minimum(by2, Y2)
        inter = jnp.maximum(xx2 - xx1, 0.0) * jnp.maximum(yy2 - yy1, 0.0)
        union = barea + AREA - inter
        iou = inter / jnp.maximum(union, 1e-9)
        S_ref[...] = jnp.where((iou > _IOU_THR) & alive, _NEG, S)
        sel = (kiota == r) & alive
        KX1 = jnp.where(sel, bx1, KX1)
        KY1 = jnp.where(sel, by1, KY1)
        KX2 = jnp.where(sel, bx2, KX2)
        KY2 = jnp.where(sel, by2, KY2)
        KS = jnp.where(sel, m, KS)
        KM = jnp.where(sel, 1, KM)
        return (KX1, KY1, KX2, KY2, KS, KM)

    init = (zK, zK, zK, zK, zK, jnp.zeros((_C, _K), jnp.int32))
    KX1, KY1, KX2, KY2, KS, KMi = lax.fori_loop(0, _K, round_body, init)
    KM = KMi > 0

    Mi = KM.astype(jnp.int32)
    run = jnp.zeros((_C, 1), jnp.int32)
    wcols = []
    for r in range(_K):
        run = run + Mi[:, r:r + 1]
        wcols.append(run)
    W = jnp.concatenate(wcols, axis=1)
    prev = jnp.zeros((1, 1), jnp.int32)
    pcols = []
    for c in range(_C):
        pcols.append(prev)
        prev = prev + run[c:c + 1, :]
    P = jnp.concatenate(pcols, axis=0)
    pos = jnp.where(KM, P + W - 1, _MAXD)

    j3 = lax.broadcasted_iota(jnp.int32, (_MAXD, _C, _K), 0)
    oh3 = (j3 == pos[None, :, :]).astype(jnp.float32)

    def scat(v):
        return jnp.sum(jnp.sum(oh3 * v[None, :, :], axis=2), axis=1,
                       keepdims=True)

    OUT = jnp.concatenate(
        [scat(KX1), scat(KY1), scat(KX2), scat(KY2), scat(KS)], axis=1)
    o_ref[0] = OUT


def kernel(pred_boxes, pred_scores, pred_cls_inds):
    B, N = pred_scores.shape
    NP = ((N + 2559) // 2560) * 2560
    bt = jnp.transpose(pred_boxes, (0, 2, 1))
    bt = jnp.pad(bt, ((0, 0), (0, 0), (0, NP - N)))
    sp = jnp.pad(pred_scores, ((0, 0), (0, NP - N)))[:, None, :]
    cp = jnp.pad(pred_cls_inds, ((0, 0), (0, NP - N)))[:, None, :]
    out = pl.pallas_call(
        _nms_body,
        grid=(B,),
        in_specs=[
            pl.BlockSpec((1, 4, NP), lambda b: (b, 0, 0)),
            pl.BlockSpec((1, 1, NP), lambda b: (b, 0, 0)),
            pl.BlockSpec((1, 1, NP), lambda b: (b, 0, 0)),
        ],
        out_specs=pl.BlockSpec((1, _MAXD, 5), lambda b: (b, 0, 0)),
        out_shape=jax.ShapeDtypeStruct((B, _MAXD, 5), jnp.float32),
        scratch_shapes=[pltpu.VMEM((_C, NP), jnp.float32)],
    )(bt, sp, cp)
    return out


# speedup vs baseline: 250.8924x; 1.6550x over previous
import functools

import jax
import jax.numpy as jnp
from jax import lax
from jax.experimental import pallas as pl
from jax.experimental.pallas import tpu as pltpu
from jax.experimental.pallas import tpu_sc as plsc

_C = 20
_K = 10
_MAXD = _C * _K
_IMG = 512.0
_SCORE_THR = 0.3
_IOU_THR = 0.5
_CAP = 2048
_CH = 2048


def _sp(x, dt):
    return jnp.zeros((16,), dt) + x


def _sc_nms(x1h, y1h, x2h, y2h, sch, clh):
    B, NPAD = sch.shape
    SENT = NPAD
    NR = NPAD + 16
    NCH = NPAD // _CH
    mesh = plsc.VectorSubcoreMesh(core_axis_name="c", subcore_axis_name="s",
                                  num_cores=2, num_subcores=16)

    @functools.partial(
        pl.kernel,
        out_type=jax.ShapeDtypeStruct((B, 2, 6 * 16 * _K), jnp.float32),
        mesh=mesh,
        compiler_params=pltpu.CompilerParams(needs_layout_passes=False),
        scratch_types=[
            pltpu.VMEM((NR,), jnp.float32),
            pltpu.VMEM((NR,), jnp.float32),
            pltpu.VMEM((NR,), jnp.float32),
            pltpu.VMEM((NR,), jnp.float32),
            pltpu.VMEM((NR,), jnp.float32),
            pltpu.VMEM((_CAP * _K + 16,), jnp.int32),
            pltpu.VMEM((_CH,), jnp.int32),
            pltpu.VMEM((6 * 16 * _K,), jnp.float32),
        ],
    )
    def k(x1_h, y1_h, x2_h, y2_h, sc_h, cl_h, out_h,
          X1, Y1, X2, Y2, SCO, LST, CLS, STG):
        b = lax.axis_index("s")
        half = lax.axis_index("c")
        cbase = half * _K
        iota16 = lax.iota(jnp.int32, 16)

        @pl.when(b < B)
        def _():
            pltpu.sync_copy(x1_h.at[b], X1.at[pl.ds(0, NPAD)])
            pltpu.sync_copy(y1_h.at[b], Y1.at[pl.ds(0, NPAD)])
            pltpu.sync_copy(x2_h.at[b], X2.at[pl.ds(0, NPAD)])
            pltpu.sync_copy(y2_h.at[b], Y2.at[pl.ds(0, NPAD)])
            pltpu.sync_copy(sc_h.at[b], SCO.at[pl.ds(0, NPAD)])
            zf = jnp.zeros((16,), jnp.float32)
            X1[pl.ds(NPAD, 16)] = zf
            Y1[pl.ds(NPAD, 16)] = zf
            X2[pl.ds(NPAD, 16)] = zf
            Y2[pl.ds(NPAD, 16)] = zf
            SCO[pl.ds(NPAD, 16)] = zf

            def clipb(i, _):
                o = i * 16
                X1[pl.ds(o, 16)] = jnp.clip(X1[pl.ds(o, 16)], 0.0, _IMG)
                Y1[pl.ds(o, 16)] = jnp.clip(Y1[pl.ds(o, 16)], 0.0, _IMG)
                X2[pl.ds(o, 16)] = jnp.clip(X2[pl.ds(o, 16)], 0.0, _IMG)
                Y2[pl.ds(o, 16)] = jnp.clip(Y2[pl.ds(o, 16)], 0.0, _IMG)
                return 0
            lax.fori_loop(0, NPAD // 16, clipb, 0)

            sentv = _sp(SENT, jnp.int32)
            def fillb(i, _):
                LST[pl.ds(i * 16, 16)] = sentv
                return 0
            lax.fori_loop(0, (_CAP * _K) // 16, fillb, 0)

            def chunkb(ch, curs):
                pltpu.sync_copy(cl_h.at[b, pl.ds(ch * _CH, _CH)], CLS)

                def vb(v, curs):
                    o = v * 16
                    go = ch * _CH + o
                    cl = CLS[pl.ds(o, 16)]
                    sv = SCO[pl.ds(go, 16)]
                    wv = X2[pl.ds(go, 16)] - X1[pl.ds(go, 16)]
                    hv = Y2[pl.ds(go, 16)] - Y1[pl.ds(go, 16)]
                    valid = (sv > _SCORE_THR) & (wv >= 0.01) & (hv >= 0.01)
                    gi = _sp(go, jnp.int32) + iota16
                    newc = []
                    for j in range(_K):
                        cj = curs[j]
                        mj = valid & (cl == (cbase + j)) & \
                            (cj <= (j * _CAP + _CAP - 16))
                        rank = plsc.cumsum(mj.astype(jnp.int32))
                        tgt = jnp.where(mj, _sp(cj, jnp.int32) + rank - 1,
                                        _sp(_CAP * _K, jnp.int32) + iota16)
                        plsc.store_scatter(LST, [tgt], gi)
                        newc.append(cj + jnp.sum(mj.astype(jnp.int32)))
                    return tuple(newc)

                return lax.fori_loop(0, _CH // 16, vb, curs)

            curs0 = tuple(jnp.int32(j * _CAP) for j in range(_K))
            curs = lax.fori_loop(0, NCH, chunkb, curs0)

            BIGP = jnp.int32(2 ** 30)
            for j in range(_K):
                base = j * _CAP
                cnt = curs[j] - base
                nv = (cnt + 15) // 16

                def roundb(r, kc):
                    KX1, KY1, KX2, KY2, KS, KM = kc

                    def amax(i, mc):
                        mv, pv = mc
                        il = LST[pl.ds(base + i * 16, 16)]
                        sv = plsc.load_gather(SCO, [il])
                        curpos = _sp(i * 16, jnp.int32) + iota16
                        gt = sv > mv
                        pv = jnp.where(gt, curpos, pv)
                        mv = jnp.where(gt, sv, mv)
                        return (mv, pv)

                    mv, pv = lax.fori_loop(
                        0, nv, amax,
                        (_sp(-1.0, jnp.float32), _sp(BIGP, jnp.int32)))
                    m = jnp.max(mv)
                    alive = m > _SCORE_THR
                    pos = jnp.min(jnp.where(mv == m, pv, BIGP))
                    safe = jnp.where(pos >= BIGP, 0, pos)
                    oi = plsc.load_gather(LST, [_sp(base, jnp.int32) +
                                                _sp(safe, jnp.int32)])
                    bx1 = plsc.load_gather(X1, [oi])
                    by1 = plsc.load_gather(Y1, [oi])
                    bx2 = plsc.load_gather(X2, [oi])
                    by2 = plsc.load_gather(Y2, [oi])
                    barea = (bx2 - bx1) * (by2 - by1)

                    def suppb(i, _):
                        sl = pl.ds(base + i * 16, 16)
                        il = LST[sl]
                        cx1 = plsc.load_gather(X1, [il])
                        cy1 = plsc.load_gather(Y1, [il])
                        cx2 = plsc.load_gather(X2, [il])
                        cy2 = plsc.load_gather(Y2, [il])
                        xx1 = jnp.maximum(bx1, cx1)
                        yy1 = jnp.maximum(by1, cy1)
                        xx2 = jnp.minimum(bx2, cx2)
                        yy2 = jnp.minimum(by2, cy2)
                        inter = jnp.maximum(xx2 - xx1, 0.0) * \
                            jnp.maximum(yy2 - yy1, 0.0)
                        carea = (cx2 - cx1) * (cy2 - cy1)
                        union = barea + carea - inter
                        iou = inter / jnp.maximum(union, 1e-9)
                        LST[sl] = jnp.where(iou > _IOU_THR, sentv, il)
                        return 0
                    lax.fori_loop(0, nv, suppb, 0)

                    sel = (iota16 == r) & alive
                    KX1 = jnp.where(sel, bx1, KX1)
                    KY1 = jnp.where(sel, by1, KY1)
                    KX2 = jnp.where(sel, bx2, KX2)
                    KY2 = jnp.where(sel, by2, KY2)
                    KS = jnp.where(sel, _sp(m, jnp.float32), KS)
                    KM = jnp.where(sel, 1.0, KM)
                    return (KX1, KY1, KX2, KY2, KS, KM)

                z = jnp.zeros((16,), jnp.float32)
                KX1, KY1, KX2, KY2, KS, KM = lax.fori_loop(
                    0, _K, roundb, (z, z, z, z, z, z))
                sb = j * 96
                STG[pl.ds(sb + 0, 16)] = KX1
                STG[pl.ds(sb + 16, 16)] = KY1
                STG[pl.ds(sb + 32, 16)] = KX2
                STG[pl.ds(sb + 48, 16)] = KY2
                STG[pl.ds(sb + 64, 16)] = KS
                STG[pl.ds(sb + 80, 16)] = KM

            pltpu.sync_copy(STG, out_h.at[b, half])

    return k(x1h, y1h, x2h, y2h, sch, clh)


def _pack_body(k_ref, o_ref):
    kk = k_ref[0]
    KX1 = kk[:, 0, 0:_K]
    KY1 = kk[:, 1, 0:_K]
    KX2 = kk[:, 2, 0:_K]
    KY2 = kk[:, 3, 0:_K]
    KS = kk[:, 4, 0:_K]
    KM = kk[:, 5, 0:_K] > 0.5

    Mi = KM.astype(jnp.int32)
    run = jnp.zeros((_C, 1), jnp.int32)
    wcols = []
    for r in range(_K):
        run = run + Mi[:, r:r + 1]
        wcols.append(run)
    W = jnp.concatenate(wcols, axis=1)
    prev = jnp.zeros((1, 1), jnp.int32)
    pcols = []
    for c in range(_C):
        pcols.append(prev)
        prev = prev + run[c:c + 1, :]
    P = jnp.concatenate(pcols, axis=0)
    pos = jnp.where(KM, P + W - 1, _MAXD)

    j3 = lax.broadcasted_iota(jnp.int32, (_MAXD, _C, _K), 0)
    oh3 = (j3 == pos[None, :, :]).astype(jnp.float32)

    def scat(v):
        return jnp.sum(jnp.sum(oh3 * v[None, :, :], axis=2), axis=1,
                       keepdims=True)

    OUT = jnp.concatenate(
        [scat(KX1), scat(KY1), scat(KX2), scat(KY2), scat(KS)], axis=1)
    o_ref[0] = OUT


def kernel(pred_boxes, pred_scores, pred_cls_inds):
    B, N = pred_scores.shape
    NPAD = ((N + _CH - 1) // _CH) * _CH
    pz = ((0, 0), (0, NPAD - N))
    x1 = jnp.pad(pred_boxes[:, :, 0], pz)
    y1 = jnp.pad(pred_boxes[:, :, 1], pz)
    x2 = jnp.pad(pred_boxes[:, :, 2], pz)
    y2 = jnp.pad(pred_boxes[:, :, 3], pz)
    sc = jnp.pad(pred_scores, pz)
    cl = jnp.pad(pred_cls_inds, pz)
    kept = _sc_nms(x1, y1, x2, y2, sc, cl)
    kept = kept.reshape(B, _C, 6, 16)
    out = pl.pallas_call(
        _pack_body,
        grid=(B,),
        in_specs=[pl.BlockSpec((1, _C, 6, 16), lambda b: (b, 0, 0, 0))],
        out_specs=pl.BlockSpec((1, _MAXD, 5), lambda b: (b, 0, 0)),
        out_shape=jax.ShapeDtypeStruct((B, _MAXD, 5), jnp.float32),
    )(kept)
    return out


# speedup vs baseline: 390.3519x; 1.5559x over previous
import functools

import jax
import jax.numpy as jnp
from jax import lax
from jax.experimental import pallas as pl
from jax.experimental.pallas import tpu as pltpu
from jax.experimental.pallas import tpu_sc as plsc

_C = 20
_K = 10
_MAXD = _C * _K
_IMG = 512.0
_SCORE_THR = 0.3
_IOU_THR = 0.5
_CAP = 2048
_CH = 2048


def _sp(x, dt):
    return jnp.zeros((16,), dt) + x


def _sc_nms(x1h, y1h, x2h, y2h, sch, clh):
    B, NPAD = sch.shape
    SENT = NPAD
    NR = NPAD + 16
    NCH = NPAD // _CH
    mesh = plsc.VectorSubcoreMesh(core_axis_name="c", subcore_axis_name="s",
                                  num_cores=2, num_subcores=16)

    @functools.partial(
        pl.kernel,
        out_type=jax.ShapeDtypeStruct((B, 2, 6 * 16 * _K), jnp.float32),
        mesh=mesh,
        compiler_params=pltpu.CompilerParams(needs_layout_passes=False),
        scratch_types=[
            pltpu.VMEM((NR,), jnp.float32),
            pltpu.VMEM((NR,), jnp.float32),
            pltpu.VMEM((NR,), jnp.float32),
            pltpu.VMEM((NR,), jnp.float32),
            pltpu.VMEM((NR,), jnp.float32),
            pltpu.VMEM((_CAP * _K + 16,), jnp.int32),
            pltpu.VMEM((_CH,), jnp.int32),
            pltpu.VMEM((6 * 16 * _K,), jnp.float32),
        ],
    )
    def k(x1_h, y1_h, x2_h, y2_h, sc_h, cl_h, out_h,
          X1, Y1, X2, Y2, SCO, LST, CLS, STG):
        b = lax.axis_index("s")
        half = lax.axis_index("c")
        cbase = half * _K
        iota16 = lax.iota(jnp.int32, 16)

        @pl.when(b < B)
        def _():
            pltpu.sync_copy(x1_h.at[b], X1.at[pl.ds(0, NPAD)])
            pltpu.sync_copy(y1_h.at[b], Y1.at[pl.ds(0, NPAD)])
            pltpu.sync_copy(x2_h.at[b], X2.at[pl.ds(0, NPAD)])
            pltpu.sync_copy(y2_h.at[b], Y2.at[pl.ds(0, NPAD)])
            pltpu.sync_copy(sc_h.at[b], SCO.at[pl.ds(0, NPAD)])
            zf = jnp.zeros((16,), jnp.float32)
            X1[pl.ds(NPAD, 16)] = zf
            Y1[pl.ds(NPAD, 16)] = zf
            X2[pl.ds(NPAD, 16)] = zf
            Y2[pl.ds(NPAD, 16)] = zf
            SCO[pl.ds(NPAD, 16)] = zf

            def clipb(i, _):
                o = i * 16
                X1[pl.ds(o, 16)] = jnp.clip(X1[pl.ds(o, 16)], 0.0, _IMG)
                Y1[pl.ds(o, 16)] = jnp.clip(Y1[pl.ds(o, 16)], 0.0, _IMG)
                X2[pl.ds(o, 16)] = jnp.clip(X2[pl.ds(o, 16)], 0.0, _IMG)
                Y2[pl.ds(o, 16)] = jnp.clip(Y2[pl.ds(o, 16)], 0.0, _IMG)
                return 0
            lax.fori_loop(0, NPAD // 16, clipb, 0)

            sentv = _sp(SENT, jnp.int32)
            def fillb(i, _):
                LST[pl.ds(i * 16, 16)] = sentv
                return 0
            lax.fori_loop(0, (_CAP * _K) // 16, fillb, 0)

            def chunkb(ch, curs):
                pltpu.sync_copy(cl_h.at[b, pl.ds(ch * _CH, _CH)], CLS)

                def vb(v, curs):
                    o = v * 16
                    go = ch * _CH + o
                    cl = CLS[pl.ds(o, 16)]
                    sv = SCO[pl.ds(go, 16)]
                    wv = X2[pl.ds(go, 16)] - X1[pl.ds(go, 16)]
                    hv = Y2[pl.ds(go, 16)] - Y1[pl.ds(go, 16)]
                    valid = (sv > _SCORE_THR) & (wv >= 0.01) & (hv >= 0.01)
                    gi = _sp(go, jnp.int32) + iota16
                    newc = []
                    for j in range(_K):
                        cj = curs[j]
                        mj = valid & (cl == (cbase + j)) & \
                            (cj <= (j * _CAP + _CAP - 16))
                        rank = plsc.cumsum(mj.astype(jnp.int32))
                        tgt = jnp.where(mj, _sp(cj, jnp.int32) + rank - 1,
                                        _sp(_CAP * _K, jnp.int32) + iota16)
                        plsc.store_scatter(LST, [tgt], gi)
                        newc.append(cj + jnp.sum(mj.astype(jnp.int32)))
                    return tuple(newc)

                return lax.fori_loop(0, _CH // 16, vb, curs)

            curs0 = tuple(jnp.int32(j * _CAP) for j in range(_K))
            curs = lax.fori_loop(0, NCH, chunkb, curs0)

            BIGP = jnp.int32(2 ** 30)
            for j in range(_K):
                base = j * _CAP
                cnt = curs[j] - base
                nv = (cnt + 15) // 16

                def roundb(r, kc):
                    KX1, KY1, KX2, KY2, KS, KM = kc

                    def amax(i, mc):
                        mv, pv = mc
                        il = LST[pl.ds(base + i * 16, 16)]
                        sv = plsc.load_gather(SCO, [il])
                        curpos = _sp(i * 16, jnp.int32) + iota16
                        gt = sv > mv
                        pv = jnp.where(gt, curpos, pv)
                        mv = jnp.where(gt, sv, mv)
                        return (mv, pv)

                    mv, pv = lax.fori_loop(
                        0, nv, amax,
                        (_sp(-1.0, jnp.float32), _sp(BIGP, jnp.int32)))
                    m = jnp.max(mv)
                    alive = m > _SCORE_THR
                    pos = jnp.min(jnp.where(mv == m, pv, BIGP))
                    safe = jnp.where(pos >= BIGP, 0, pos)
                    oi = plsc.load_gather(LST, [_sp(base, jnp.int32) +
                                                _sp(safe, jnp.int32)])
                    bx1 = plsc.load_gather(X1, [oi])
                    by1 = plsc.load_gather(Y1, [oi])
                    bx2 = plsc.load_gather(X2, [oi])
                    by2 = plsc.load_gather(Y2, [oi])
                    barea = (bx2 - bx1) * (by2 - by1)

                    def suppb(i, _):
                        sl = pl.ds(base + i * 16, 16)
                        il = LST[sl]
                        cx1 = plsc.load_gather(X1, [il])
                        cy1 = plsc.load_gather(Y1, [il])
                        cx2 = plsc.load_gather(X2, [il])
                        cy2 = plsc.load_gather(Y2, [il])
                        xx1 = jnp.maximum(bx1, cx1)
                        yy1 = jnp.maximum(by1, cy1)
                        xx2 = jnp.minimum(bx2, cx2)
                        yy2 = jnp.minimum(by2, cy2)
                        inter = jnp.maximum(xx2 - xx1, 0.0) * \
                            jnp.maximum(yy2 - yy1, 0.0)
                        carea = (cx2 - cx1) * (cy2 - cy1)
                        union = barea + carea - inter
                        iou = inter / jnp.maximum(union, 1e-9)
                        LST[sl] = jnp.where(iou > _IOU_THR, sentv, il)
                        return 0
                    lax.fori_loop(0, nv, suppb, 0)

                    sel = (iota16 == r) & alive
                    KX1 = jnp.where(sel, bx1, KX1)
                    KY1 = jnp.where(sel, by1, KY1)
                    KX2 = jnp.where(sel, bx2, KX2)
                    KY2 = jnp.where(sel, by2, KY2)
                    KS = jnp.where(sel, _sp(m, jnp.float32), KS)
                    KM = jnp.where(sel, 1.0, KM)
                    return (KX1, KY1, KX2, KY2, KS, KM)

                z = jnp.zeros((16,), jnp.float32)
                KX1, KY1, KX2, KY2, KS, KM = lax.fori_loop(
                    0, _K, roundb, (z, z, z, z, z, z))
                sb = j * 96
                STG[pl.ds(sb + 0, 16)] = KX1
                STG[pl.ds(sb + 16, 16)] = KY1
                STG[pl.ds(sb + 32, 16)] = KX2
                STG[pl.ds(sb + 48, 16)] = KY2
                STG[pl.ds(sb + 64, 16)] = KS
                STG[pl.ds(sb + 80, 16)] = KM

            pltpu.sync_copy(STG, out_h.at[b, half])

    return k(x1h, y1h, x2h, y2h, sch, clh)


def _pack_body(vr_ref, vc_ref, o_ref):
    vr = vr_ref[0]
    vc = vc_ref[0]
    Mrow = vr[5:6, :]
    ii = lax.broadcasted_iota(jnp.int32, (_MAXD, _MAXD), 0)
    jj = lax.broadcasted_iota(jnp.int32, (_MAXD, _MAXD), 1)
    triu = (ii <= jj).astype(jnp.float32)
    posrow = jnp.dot(Mrow, triu,
                     preferred_element_type=jnp.float32)
    posi = posrow.astype(jnp.int32) - 1
    oh = ((ii == posi) & (Mrow > 0.5)).astype(jnp.float32)
    out6 = jnp.dot(oh, vc, preferred_element_type=jnp.float32)
    o_ref[0] = out6[:, 0:5]


def kernel(pred_boxes, pred_scores, pred_cls_inds):
    B, N = pred_scores.shape
    NPAD = ((N + _CH - 1) // _CH) * _CH
    pz = ((0, 0), (0, NPAD - N))
    x1 = jnp.pad(pred_boxes[:, :, 0], pz)
    y1 = jnp.pad(pred_boxes[:, :, 1], pz)
    x2 = jnp.pad(pred_boxes[:, :, 2], pz)
    y2 = jnp.pad(pred_boxes[:, :, 3], pz)
    sc = jnp.pad(pred_scores, pz)
    cl = jnp.pad(pred_cls_inds, pz)
    kept = _sc_nms(x1, y1, x2, y2, sc, cl)
    k5 = kept.reshape(B, 2, _K, 6, 16)[:, :, :, :, :_K]
    vrow = k5.transpose(0, 3, 1, 2, 4).reshape(B, 6, _MAXD)
    vcol = vrow.transpose(0, 2, 1)
    out = pl.pallas_call(
        _pack_body,
        grid=(B,),
        in_specs=[
            pl.BlockSpec((1, 6, _MAXD), lambda b: (b, 0, 0)),
            pl.BlockSpec((1, _MAXD, 6), lambda b: (b, 0, 0)),
        ],
        out_specs=pl.BlockSpec((1, _MAXD, 5), lambda b: (b, 0, 0)),
        out_shape=jax.ShapeDtypeStruct((B, _MAXD, 5), jnp.float32),
    )(vrow, vcol)
    return out
